# baseline (speedup 1.0000x reference)
import functools

import jax
import jax.numpy as jnp
from jax import lax
from jax.experimental import pallas as pl
from jax.experimental.pallas import tpu as pltpu
from jax.experimental.pallas import tpu_sc as plsc

_BETA2 = 1.5 * 1.5
_COEF = 1.0 + _BETA2

_NSC = 1
_NT = 32
_RCH = 4


def _tc_body(x_ref, t_ref, out_ref):
    b = pl.program_id(0)
    h = pl.program_id(1)
    nb = pl.num_programs(0)
    nh = pl.num_programs(1)

    x = x_ref[0]
    t = t_ref[0]
    th = jnp.tanh(x * 0.5)
    cls = jax.lax.broadcasted_iota(jnp.int32, x.shape, 0)
    s_part = jnp.sum(th, axis=(0, 1))
    tp_part = jnp.sum(jnp.where(cls == t[None], th, 0.0), axis=(0, 1))

    @pl.when((b == 0) & (h == 0))
    def _():
        out_ref[...] = jnp.zeros(out_ref.shape, out_ref.dtype)

    row = jax.lax.broadcasted_iota(jnp.int32, out_ref.shape[1:], 0)
    hit = row == b
    out_ref[0] += jnp.where(hit, tp_part[None, :], 0.0)
    out_ref[1] += jnp.where(hit, s_part[None, :], 0.0)


def _sc_body(yp, yt, out, xbuf, tbuf, otp, osum, semx, semt, *,
             b0, nsc, c, h, w, rows_per_tile):
    wid = lax.axis_index("s") * 2 + lax.axis_index("c")
    nch = rows_per_tile // _RCH
    nkv = w // 16
    lanes = lax.iota(jnp.int32, 16)

    def start(i):
        bi, ch = i // nch, i % nch
        slot = i % 2
        h0 = wid * rows_per_tile + ch * _RCH
        cx = pltpu.async_copy(
            yp.at[b0 + bi, :, pl.ds(h0, _RCH), :], xbuf.at[slot], semx)
        ct = pltpu.async_copy(
            yt.at[b0 + bi, pl.ds(h0, _RCH), :], tbuf.at[slot], semt)
        return cx, ct

    pend = start(0)
    for bi in range(nsc):
        acc_tp = jnp.zeros((16,), jnp.float32)
        acc_s = jnp.zeros((16,), jnp.float32)
        for ch in range(nch):
            i = bi * nch + ch
            slot = i % 2
            if i + 1 < nsc * nch:
                nxt = start(i + 1)
            else:
                nxt = None
            pend[0].wait()
            pend[1].wait()
            pend = nxt
            xr = xbuf.at[slot]
            tr = tbuf.at[slot]

            def srow(r, accs):
                cc = r // _RCH
                jj = r % _RCH
                a = list(accs)
                for k in range(nkv):
                    v = xr[cc, jj, pl.ds(k * 16, 16)]
                    a[k % 4] = a[k % 4] + 1.0 / (1.0 + jnp.exp(-v))
                return tuple(a)
            z = jnp.zeros((16,), jnp.float32)
            accs = lax.fori_loop(0, c * _RCH, srow, (z, z, z, z))
            acc_s = acc_s + accs[0] + accs[1] + accs[2] + accs[3]

            for jj in range(_RCH):
                jv = jnp.full((16,), jj, jnp.int32)
                def gkv(k4, accs2):
                    a = list(accs2)
                    for u in range(4):
                        k = k4 * 4 + u
                        t16 = tr[jj, pl.ds(k * 16, 16)]
                        wv = k * 16 + lanes
                        v = plsc.load_gather(xr, [t16, jv, wv])
                        a[u] = a[u] + 1.0 / (1.0 + jnp.exp(-v))
                    return tuple(a)
                accs2 = lax.fori_loop(0, nkv // 4, gkv, (z, z, z, z))
                acc_tp = acc_tp + accs2[0] + accs2[1] + accs2[2] + accs2[3]
        otp[...] = acc_tp
        osum[...] = acc_s
        pltpu.sync_copy(otp, out.at[0, bi, pl.ds(wid * 16, 16)])
        pltpu.sync_copy(osum, out.at[1, bi, pl.ds(wid * 16, 16)])


def _combine_body(tc_ref, sc_ref, out_ref, *, hw, c, nbtc, nsc):
    tp_tc = 0.5 * hw + 0.5 * jnp.sum(tc_ref[0], axis=1)
    s_tc = 0.5 * (c * hw) + 0.5 * jnp.sum(tc_ref[1], axis=1)
    f_tc = _COEF * tp_tc / (s_tc + _BETA2 * hw)
    tp_sc = jnp.sum(sc_ref[0], axis=1)
    s_sc = jnp.sum(sc_ref[1], axis=1)
    f_sc = _COEF * tp_sc / (s_sc + _BETA2 * hw)
    tot = jnp.sum(1.0 - f_tc) + jnp.sum(1.0 - f_sc)
    out_ref[...] = (tot / (nbtc + nsc))[None, None]


def kernel(y_pred, y_true):
    B, C, H, W = y_pred.shape
    nsc = _NSC
    nbtc = B - nsc
    HB = 256
    nh = H // HB
    rows_per_tile = H // _NT

    mesh = plsc.VectorSubcoreMesh(core_axis_name="c", subcore_axis_name="s")
    sc_part = functools.partial(
        pl.kernel,
        mesh=mesh,
        compiler_params=pltpu.CompilerParams(
            needs_layout_passes=False, vmem_limit_bytes=1 << 20),
        out_type=jax.ShapeDtypeStruct((2, nsc, _NT * 16), jnp.float32),
        scratch_types=[
            pltpu.VMEM((2, C, _RCH, W), jnp.float32),
            pltpu.VMEM((2, _RCH, W), jnp.int32),
            pltpu.VMEM((16,), jnp.float32),
            pltpu.VMEM((16,), jnp.float32),
            pltpu.SemaphoreType.DMA,
            pltpu.SemaphoreType.DMA,
        ],
    )(functools.partial(
        _sc_body, b0=nbtc, nsc=nsc, c=C, h=H, w=W,
        rows_per_tile=rows_per_tile,
    ))(y_pred, y_true)

    tc_part = pl.pallas_call(
        _tc_body,
        grid=(nbtc, nh),
        in_specs=[
            pl.BlockSpec((1, C, HB, W), lambda b, h: (b, 0, h, 0)),
            pl.BlockSpec((1, HB, W), lambda b, h: (b, h, 0)),
        ],
        out_specs=pl.BlockSpec((2, nbtc, W), lambda b, h: (0, 0, 0)),
        out_shape=jax.ShapeDtypeStruct((2, nbtc, W), jnp.float32),
    )(y_pred, y_true)

    out = pl.pallas_call(
        functools.partial(
            _combine_body, hw=float(H * W), c=C, nbtc=nbtc, nsc=nsc),
        out_shape=jax.ShapeDtypeStruct((1, 1), jnp.float32),
    )(tc_part, sc_part)
    return out[0, 0]


# speedup vs baseline: 1.0104x; 1.0104x over previous
import functools

import jax
import jax.numpy as jnp
from jax import lax
from jax.experimental import pallas as pl
from jax.experimental.pallas import tpu as pltpu
from jax.experimental.pallas import tpu_sc as plsc

_BETA2 = 1.5 * 1.5
_COEF = 1.0 + _BETA2

_NSC = 1
_NT = 32
_RCH = 4


def _tc_body(x_ref, t_ref, out_ref):
    b = pl.program_id(0)
    h = pl.program_id(1)
    nb = pl.num_programs(0)
    nh = pl.num_programs(1)

    x = x_ref[0]
    t = t_ref[0]
    th = jnp.tanh(x * 0.5)
    cls = jax.lax.broadcasted_iota(jnp.int32, x.shape, 0)
    s_part = jnp.sum(th, axis=(0, 1))
    tp_part = jnp.sum(jnp.where(cls == t[None], th, 0.0), axis=(0, 1))

    @pl.when((b == 0) & (h == 0))
    def _():
        out_ref[...] = jnp.zeros(out_ref.shape, out_ref.dtype)

    row = jax.lax.broadcasted_iota(jnp.int32, out_ref.shape[1:], 0)
    hit = row == b
    out_ref[0] += jnp.where(hit, tp_part[None, :], 0.0)
    out_ref[1] += jnp.where(hit, s_part[None, :], 0.0)


def _sc_body(yp, yt, out, xbuf, tbuf, otp, osum, semx, semt, *,
             b0, nsc, c, h, w, rows_per_tile):
    wid = lax.axis_index("s") * 2 + lax.axis_index("c")
    nch = rows_per_tile // _RCH
    nkv = w // 16
    lanes = lax.iota(jnp.int32, 16)

    def start(i):
        bi, ch = i // nch, i % nch
        slot = i % 2
        h0 = wid * rows_per_tile + ch * _RCH
        cx = pltpu.async_copy(
            yp.at[b0 + bi, :, pl.ds(h0, _RCH), :], xbuf.at[slot], semx)
        ct = pltpu.async_copy(
            yt.at[b0 + bi, pl.ds(h0, _RCH), :], tbuf.at[slot], semt)
        return cx, ct

    pend = start(0)
    for bi in range(nsc):
        acc_tp = jnp.zeros((16,), jnp.float32)
        acc_s = jnp.zeros((16,), jnp.float32)
        for ch in range(nch):
            i = bi * nch + ch
            slot = i % 2
            if i + 1 < nsc * nch:
                nxt = start(i + 1)
            else:
                nxt = None
            pend[0].wait()
            pend[1].wait()
            pend = nxt
            xr = xbuf.at[slot]
            tr = tbuf.at[slot]

            def srow(r, accs):
                cc = r // _RCH
                jj = r % _RCH
                a = list(accs)
                for k in range(nkv):
                    v = xr[cc, jj, pl.ds(k * 16, 16)]
                    a[k % 4] = a[k % 4] + 1.0 / (1.0 + jnp.exp(-v))
                return tuple(a)
            z = jnp.zeros((16,), jnp.float32)
            accs = lax.fori_loop(0, c * _RCH, srow, (z, z, z, z))
            acc_s = acc_s + accs[0] + accs[1] + accs[2] + accs[3]

            for jj in range(_RCH):
                jv = jnp.full((16,), jj, jnp.int32)
                def gkv(k4, accs2):
                    a = list(accs2)
                    for u in range(4):
                        k = k4 * 4 + u
                        t16 = tr[jj, pl.ds(k * 16, 16)]
                        wv = k * 16 + lanes
                        v = plsc.load_gather(xr, [t16, jv, wv])
                        a[u] = a[u] + 1.0 / (1.0 + jnp.exp(-v))
                    return tuple(a)
                accs2 = lax.fori_loop(0, nkv // 4, gkv, (z, z, z, z))
                acc_tp = acc_tp + accs2[0] + accs2[1] + accs2[2] + accs2[3]
        otp[...] = acc_tp
        osum[...] = acc_s
        pltpu.sync_copy(otp, out.at[0, bi, pl.ds(wid * 16, 16)])
        pltpu.sync_copy(osum, out.at[1, bi, pl.ds(wid * 16, 16)])


def _combine_body(tc_ref, sc_ref, out_ref, *, hw, c, nbtc, nsc):
    tp_tc = 0.5 * hw + 0.5 * jnp.sum(tc_ref[0], axis=1)
    s_tc = 0.5 * (c * hw) + 0.5 * jnp.sum(tc_ref[1], axis=1)
    f_tc = _COEF * tp_tc / (s_tc + _BETA2 * hw)
    tp_sc = jnp.sum(sc_ref[0], axis=1)
    s_sc = jnp.sum(sc_ref[1], axis=1)
    f_sc = _COEF * tp_sc / (s_sc + _BETA2 * hw)
    tot = jnp.sum(1.0 - f_tc) + jnp.sum(1.0 - f_sc)
    out_ref[...] = (tot / (nbtc + nsc))[None, None]


def kernel(y_pred, y_true):
    B, C, H, W = y_pred.shape
    nsc = _NSC
    nbtc = B - nsc
    HB = 256
    nh = H // HB
    rows_per_tile = H // _NT

    mesh = plsc.VectorSubcoreMesh(core_axis_name="c", subcore_axis_name="s")
    sc_part = functools.partial(
        pl.kernel,
        mesh=mesh,
        compiler_params=pltpu.CompilerParams(
            needs_layout_passes=False, vmem_limit_bytes=1 << 20),
        out_type=jax.ShapeDtypeStruct((2, nsc, _NT * 16), jnp.float32),
        scratch_types=[
            pltpu.VMEM((2, C, _RCH, W), jnp.float32),
            pltpu.VMEM((2, _RCH, W), jnp.int32),
            pltpu.VMEM((16,), jnp.float32),
            pltpu.VMEM((16,), jnp.float32),
            pltpu.SemaphoreType.DMA,
            pltpu.SemaphoreType.DMA,
        ],
    )(functools.partial(
        _sc_body, b0=nbtc, nsc=nsc, c=C, h=H, w=W,
        rows_per_tile=rows_per_tile,
    ))(y_pred, y_true)

    tc_part = pl.pallas_call(
        _tc_body,
        grid=(nbtc, nh),
        in_specs=[
            pl.BlockSpec((1, C, HB, W), lambda b, h: (b, 0, h, 0)),
            pl.BlockSpec((1, HB, W), lambda b, h: (b, h, 0)),
        ],
        out_specs=pl.BlockSpec((2, nbtc, W), lambda b, h: (0, 0, 0)),
        out_shape=jax.ShapeDtypeStruct((2, nbtc, W), jnp.float32),
        compiler_params=pltpu.CompilerParams(vmem_limit_bytes=48 << 20),
    )(y_pred, y_true)

    out = pl.pallas_call(
        functools.partial(
            _combine_body, hw=float(H * W), c=C, nbtc=nbtc, nsc=nsc),
        out_shape=jax.ShapeDtypeStruct((1, 1), jnp.float32),
        compiler_params=pltpu.CompilerParams(vmem_limit_bytes=1 << 20),
    )(tc_part, sc_part)
    return out[0, 0]


# speedup vs baseline: 1.3013x; 1.2880x over previous
import functools

import jax
import jax.numpy as jnp
from jax.experimental import pallas as pl
from jax.experimental.pallas import tpu as pltpu

_BETA2 = 1.5 * 1.5
_COEF = 1.0 + _BETA2


def _tc_body(x_ref, t_ref, out_ref, tp_acc, s_acc, *, hw):
    b = pl.program_id(0)
    h = pl.program_id(1)
    nb = pl.num_programs(0)
    nh = pl.num_programs(1)

    x = x_ref[0]
    t = t_ref[0]
    th = jnp.tanh(x * 0.5)
    cls = jax.lax.broadcasted_iota(jnp.int32, x.shape, 0)
    s_part = jnp.sum(th, axis=(0, 1))
    tp_part = jnp.sum(jnp.where(cls == t[None], th, 0.0), axis=(0, 1))

    @pl.when((b == 0) & (h == 0))
    def _():
        tp_acc[...] = jnp.zeros(tp_acc.shape, tp_acc.dtype)
        s_acc[...] = jnp.zeros(s_acc.shape, s_acc.dtype)

    row = jax.lax.broadcasted_iota(jnp.int32, tp_acc.shape, 0)
    hit = row == b
    tp_acc[...] += jnp.where(hit, tp_part[None, :], 0.0)
    s_acc[...] += jnp.where(hit, s_part[None, :], 0.0)

    @pl.when((b == nb - 1) & (h == nh - 1))
    def _():
        c = x_ref.shape[1]
        tp = 0.5 * hw + 0.5 * jnp.sum(tp_acc[...], axis=1)
        s = 0.5 * (c * hw) + 0.5 * jnp.sum(s_acc[...], axis=1)
        f = _COEF * tp / (s + _BETA2 * hw)
        out_ref[...] = jnp.mean(1.0 - f)[None, None]


def kernel(y_pred, y_true):
    B, C, H, W = y_pred.shape
    HB = 256
    nh = H // HB
    out = pl.pallas_call(
        functools.partial(_tc_body, hw=float(H * W)),
        grid=(B, nh),
        in_specs=[
            pl.BlockSpec((1, C, HB, W), lambda b, h: (b, 0, h, 0)),
            pl.BlockSpec((1, HB, W), lambda b, h: (b, h, 0)),
        ],
        out_specs=pl.BlockSpec((1, 1), lambda b, h: (0, 0)),
        out_shape=jax.ShapeDtypeStruct((1, 1), jnp.float32),
        scratch_shapes=[
            pltpu.VMEM((B, W), jnp.float32),
            pltpu.VMEM((B, W), jnp.float32),
        ],
    )(y_pred, y_true)
    return out[0, 0]
